# Initial kernel scaffold; baseline (speedup 1.0000x reference)
import jax
import jax.numpy as jnp
from jax.experimental import pallas as pl


def kernel(x, edge_index, edge_attr, u, batch, mw1, mb1, mw2, mb2, mw3, mb3, nw1, nb1, nw2, nb2, nw3, nb3):
    raise NotImplementedError("write your pallas kernel here")


# speedup vs baseline: 2.4384x; 2.4384x over previous
import functools

import jax
import jax.numpy as jnp
from jax import lax
from jax.experimental import pallas as pl
from jax.experimental.pallas import tpu as pltpu
from jax.experimental.pallas import tpu_sc as plsc

N = 10000
E = 320000
DF = 128
DH = 16

NC = 2
NS = 16
NW = NC * NS
EPW = E // NW
CB = 80
NCH = EPW // CB
RPT = N // NS


def _xs_body(x_ref, w_ref, o_ref):
    o_ref[...] = jnp.dot(x_ref[...], w_ref[...].T,
                         preferred_element_type=jnp.float32)


def _node_pre(x, mw1a):
    bm = 2000
    return pl.pallas_call(
        _xs_body,
        grid=(N // bm,),
        in_specs=[pl.BlockSpec((bm, DF), lambda i: (i, 0)),
                  pl.BlockSpec((DH, DF), lambda i: (0, 0))],
        out_specs=pl.BlockSpec((bm, DH), lambda i: (i, 0)),
        out_shape=jax.ShapeDtypeStruct((N, DH), jnp.float32),
    )(x, mw1a)


def _edge_body(xg_ref, ea_ref, w1b_ref, b1_ref, w2_ref, b2_ref, w3_ref,
               b3_ref, o_ref):
    t = xg_ref[...] + jnp.dot(ea_ref[...], w1b_ref[...].T,
                              preferred_element_type=jnp.float32) + b1_ref[...]
    t = jnp.maximum(t, 0.0)
    t = jnp.dot(t, w2_ref[...].T, preferred_element_type=jnp.float32) + b2_ref[...]
    t = jnp.maximum(t, 0.0)
    o_ref[...] = jnp.dot(t, w3_ref[...].T,
                         preferred_element_type=jnp.float32) + b3_ref[...]


def _edge_mlp(xg, ea, w1b, b1, w2, b2, w3, b3):
    bm = 8000
    wspec = pl.BlockSpec((DH, DH), lambda i: (0, 0))
    bspec = pl.BlockSpec((1, DH), lambda i: (0, 0))
    return pl.pallas_call(
        _edge_body,
        grid=(E // bm,),
        in_specs=[pl.BlockSpec((bm, DH), lambda i: (i, 0)),
                  pl.BlockSpec((bm, DH), lambda i: (i, 0)),
                  wspec, bspec, wspec, bspec, wspec, bspec],
        out_specs=pl.BlockSpec((bm, DH), lambda i: (i, 0)),
        out_shape=jax.ShapeDtypeStruct((E, DH), jnp.float32),
    )(xg, ea, w1b, b1, w2, b2, w3, b3)


def _node_body(x_ref, p0_ref, p1_ref, w1a_ref, w1b_ref, b1_ref, w2_ref,
               b2_ref, w3_ref, b3_ref, o_ref):
    agg = p0_ref[...] + p1_ref[...]
    t = (jnp.dot(x_ref[...], w1a_ref[...].T, preferred_element_type=jnp.float32)
         + jnp.dot(agg, w1b_ref[...].T, preferred_element_type=jnp.float32)
         + b1_ref[...])
    t = jnp.maximum(t, 0.0)
    t = jnp.dot(t, w2_ref[...].T, preferred_element_type=jnp.float32) + b2_ref[...]
    t = jnp.maximum(t, 0.0)
    o_ref[...] = jnp.dot(t, w3_ref[...].T,
                         preferred_element_type=jnp.float32) + b3_ref[...]


def _node_mlp(x, parts, w1a, w1b, b1, w2, b2, w3, b3):
    bm = 2000
    nb = N // bm
    wspec = pl.BlockSpec((DH, DH), lambda i: (0, 0))
    bspec = pl.BlockSpec((1, DH), lambda i: (0, 0))
    return pl.pallas_call(
        _node_body,
        grid=(nb,),
        in_specs=[pl.BlockSpec((bm, DF), lambda i: (i, 0)),
                  pl.BlockSpec((bm, DH), lambda i: (i, 0)),
                  pl.BlockSpec((bm, DH), lambda i: (i + nb, 0)),
                  pl.BlockSpec((DH, DF), lambda i: (0, 0)),
                  wspec, bspec, wspec, bspec, wspec, bspec],
        out_specs=pl.BlockSpec((bm, DH), lambda i: (i, 0)),
        out_shape=jax.ShapeDtypeStruct((N, DH), jnp.float32),
    )(x, parts, parts, w1a, w1b, b1, w2, b2, w3, b3)


def _gather_sc(xs, send):
    mesh = plsc.VectorSubcoreMesh(core_axis_name="c", subcore_axis_name="s")

    @functools.partial(
        pl.kernel,
        out_type=jax.ShapeDtypeStruct((E, DH), jnp.float32),
        mesh=mesh,
        scratch_types=[pltpu.VMEM((CB,), jnp.int32),
                       pltpu.VMEM((CB, DH), jnp.float32),
                       pltpu.SemaphoreType.DMA],
        compiler_params=pltpu.CompilerParams(use_tc_tiling_on_sc=False),
    )
    def k(xs_hbm, send_hbm, out_hbm, idx_v, rows_v, sem):
        wid = lax.axis_index("s") * NC + lax.axis_index("c")
        base = wid * EPW

        def body(j, carry):
            off = base + j * CB
            pltpu.sync_copy(send_hbm.at[pl.ds(off, CB)], idx_v)
            pltpu.async_copy(xs_hbm.at[idx_v], rows_v, sem).wait()
            pltpu.sync_copy(rows_v, out_hbm.at[pl.ds(off, CB)])
            return carry

        lax.fori_loop(0, NCH, body, 0)

    return k(xs, send)


def _scatter_sc(m3, rec):
    mesh = plsc.VectorSubcoreMesh(core_axis_name="c", subcore_axis_name="s")

    @functools.partial(
        pl.kernel,
        out_type=jax.ShapeDtypeStruct((NC * N, DH), jnp.float32),
        mesh=mesh,
        scratch_types=[pltpu.VMEM_SHARED((N, DH), jnp.float32),
                       pltpu.VMEM((RPT, DH), jnp.float32),
                       pltpu.VMEM((CB, DH), jnp.float32),
                       pltpu.VMEM((CB,), jnp.int32),
                       pltpu.SemaphoreType.DMA],
        compiler_params=pltpu.CompilerParams(use_tc_tiling_on_sc=False),
    )
    def k(m3_hbm, rec_hbm, out_hbm, agg_sh, zrows_v, m3_v, rec_v, sem):
        cid = lax.axis_index("c")
        sid = lax.axis_index("s")
        wid = sid * NC + cid

        def zero_body(r, carry):
            zrows_v[r, :] = jnp.zeros((DH,), jnp.float32)
            return carry

        lax.fori_loop(0, RPT, zero_body, 0)
        pltpu.sync_copy(zrows_v, agg_sh.at[pl.ds(sid * RPT, RPT)])
        plsc.subcore_barrier()

        base = wid * EPW

        def body(j, carry):
            off = base + j * CB
            pltpu.sync_copy(rec_hbm.at[pl.ds(off, CB)], rec_v)
            pltpu.sync_copy(m3_hbm.at[pl.ds(off, CB)], m3_v)
            pltpu.sync_copy(m3_v, agg_sh.at[rec_v], add=True)
            return carry

        lax.fori_loop(0, NCH, body, 0)
        plsc.subcore_barrier()
        pltpu.sync_copy(agg_sh.at[pl.ds(sid * RPT, RPT)],
                        out_hbm.at[pl.ds(cid * N + sid * RPT, RPT)])

    return k(m3, rec)


def kernel(x, edge_index, edge_attr, u, batch, mw1, mb1, mw2, mb2, mw3, mb3,
           nw1, nb1, nw2, nb2, nw3, nb3):
    send = edge_index[0]
    rec = edge_index[1]
    mw1a = mw1[:, :DF]
    mw1b = mw1[:, DF:]
    nw1a = nw1[:, :DF]
    nw1b = nw1[:, DF:]
    b1 = mb1.reshape(1, DH)
    b2 = mb2.reshape(1, DH)
    b3 = mb3.reshape(1, DH)
    c1 = nb1.reshape(1, DH)
    c2 = nb2.reshape(1, DH)
    c3 = nb3.reshape(1, DH)

    xs = _node_pre(x, mw1a)
    xg = _gather_sc(xs, send)
    m3 = _edge_mlp(xg, edge_attr, mw1b, b1, mw2, b2, mw3, b3)
    parts = _scatter_sc(m3, rec)
    h = _node_mlp(x, parts, nw1a, nw1b, c1, nw2, c2, nw3, c3)
    return h


# speedup vs baseline: 3.3740x; 1.3837x over previous
import functools

import jax
import jax.numpy as jnp
from jax import lax
from jax.experimental import pallas as pl
from jax.experimental.pallas import tpu as pltpu
from jax.experimental.pallas import tpu_sc as plsc

N = 10000
E = 320000
DF = 128
DH = 16

NC = 2
NS = 16
NW = NC * NS
EPW = E // NW
CB = 1000
NCH = EPW // CB
NBUF = 4
RPT = N // NS


def _xs_body(x_ref, w_ref, o_ref):
    o_ref[...] = jnp.dot(x_ref[...], w_ref[...].T,
                         preferred_element_type=jnp.float32)


def _node_pre(x, mw1a):
    bm = 2000
    return pl.pallas_call(
        _xs_body,
        grid=(N // bm,),
        in_specs=[pl.BlockSpec((bm, DF), lambda i: (i, 0)),
                  pl.BlockSpec((DH, DF), lambda i: (0, 0))],
        out_specs=pl.BlockSpec((bm, DH), lambda i: (i, 0)),
        out_shape=jax.ShapeDtypeStruct((N, DH), jnp.float32),
    )(x, mw1a)


def _edge_body(xg_ref, ea_ref, w1b_ref, b1_ref, w2_ref, b2_ref, w3_ref,
               b3_ref, o_ref):
    t = xg_ref[...] + jnp.dot(ea_ref[...], w1b_ref[...].T,
                              preferred_element_type=jnp.float32) + b1_ref[...]
    t = jnp.maximum(t, 0.0)
    t = jnp.dot(t, w2_ref[...].T, preferred_element_type=jnp.float32) + b2_ref[...]
    t = jnp.maximum(t, 0.0)
    o_ref[...] = jnp.dot(t, w3_ref[...].T,
                         preferred_element_type=jnp.float32) + b3_ref[...]


def _edge_mlp(xg, ea, w1b, b1, w2, b2, w3, b3):
    bm = 8000
    wspec = pl.BlockSpec((DH, DH), lambda i: (0, 0))
    bspec = pl.BlockSpec((1, DH), lambda i: (0, 0))
    return pl.pallas_call(
        _edge_body,
        grid=(E // bm,),
        in_specs=[pl.BlockSpec((bm, DH), lambda i: (i, 0)),
                  pl.BlockSpec((bm, DH), lambda i: (i, 0)),
                  wspec, bspec, wspec, bspec, wspec, bspec],
        out_specs=pl.BlockSpec((bm, DH), lambda i: (i, 0)),
        out_shape=jax.ShapeDtypeStruct((E, DH), jnp.float32),
    )(xg, ea, w1b, b1, w2, b2, w3, b3)


def _node_body(x_ref, p0_ref, p1_ref, w1a_ref, w1b_ref, b1_ref, w2_ref,
               b2_ref, w3_ref, b3_ref, o_ref):
    agg = p0_ref[...] + p1_ref[...]
    t = (jnp.dot(x_ref[...], w1a_ref[...].T, preferred_element_type=jnp.float32)
         + jnp.dot(agg, w1b_ref[...].T, preferred_element_type=jnp.float32)
         + b1_ref[...])
    t = jnp.maximum(t, 0.0)
    t = jnp.dot(t, w2_ref[...].T, preferred_element_type=jnp.float32) + b2_ref[...]
    t = jnp.maximum(t, 0.0)
    o_ref[...] = jnp.dot(t, w3_ref[...].T,
                         preferred_element_type=jnp.float32) + b3_ref[...]


def _node_mlp(x, parts, w1a, w1b, b1, w2, b2, w3, b3):
    bm = 2000
    nb = N // bm
    wspec = pl.BlockSpec((DH, DH), lambda i: (0, 0))
    bspec = pl.BlockSpec((1, DH), lambda i: (0, 0))
    return pl.pallas_call(
        _node_body,
        grid=(nb,),
        in_specs=[pl.BlockSpec((bm, DF), lambda i: (i, 0)),
                  pl.BlockSpec((bm, DH), lambda i: (i, 0)),
                  pl.BlockSpec((bm, DH), lambda i: (i + nb, 0)),
                  pl.BlockSpec((DH, DF), lambda i: (0, 0)),
                  wspec, bspec, wspec, bspec, wspec, bspec],
        out_specs=pl.BlockSpec((bm, DH), lambda i: (i, 0)),
        out_shape=jax.ShapeDtypeStruct((N, DH), jnp.float32),
    )(x, parts, parts, w1a, w1b, b1, w2, b2, w3, b3)


def _gather_sc(xs, send):
    mesh = plsc.VectorSubcoreMesh(core_axis_name="c", subcore_axis_name="s")

    @functools.partial(
        pl.kernel,
        out_type=jax.ShapeDtypeStruct((E, DH), jnp.float32),
        mesh=mesh,
        scratch_types=[pltpu.VMEM((EPW,), jnp.int32),
                       [pltpu.VMEM((CB, DH), jnp.float32)] * NBUF,
                       [pltpu.SemaphoreType.DMA] * NBUF,
                       [pltpu.SemaphoreType.DMA] * NBUF],
        compiler_params=pltpu.CompilerParams(use_tc_tiling_on_sc=False),
    )
    def k(xs_hbm, send_hbm, out_hbm, idx_v, rows, gsem, wsem):
        wid = lax.axis_index("s") * NC + lax.axis_index("c")
        base = wid * EPW
        pltpu.sync_copy(send_hbm.at[pl.ds(base, EPW)], idx_v)

        gd = [None] * NBUF
        wd = [None] * NBUF
        for j in range(min(NBUF, NCH)):
            gd[j] = pltpu.async_copy(
                xs_hbm.at[idx_v.at[pl.ds(j * CB, CB)]], rows[j], gsem[j])
        for j in range(NCH):
            b = j % NBUF
            gd[b].wait()
            wd[b] = pltpu.async_copy(
                rows[b], out_hbm.at[pl.ds(base + j * CB, CB)], wsem[b])
            nj = j + NBUF
            if nj < NCH:
                wd[b].wait()
                gd[b] = pltpu.async_copy(
                    xs_hbm.at[idx_v.at[pl.ds(nj * CB, CB)]], rows[b], gsem[b])
        for j in range(max(0, NCH - NBUF), NCH):
            wd[j % NBUF].wait()

    return k(xs, send)


def _scatter_sc(m3, rec):
    mesh = plsc.VectorSubcoreMesh(core_axis_name="c", subcore_axis_name="s")

    @functools.partial(
        pl.kernel,
        out_type=jax.ShapeDtypeStruct((NC * N, DH), jnp.float32),
        mesh=mesh,
        scratch_types=[pltpu.VMEM_SHARED((N, DH), jnp.float32),
                       pltpu.VMEM((RPT, DH), jnp.float32),
                       pltpu.VMEM((NCH, CB), jnp.int32),
                       [pltpu.VMEM((CB, DH), jnp.float32)] * NBUF,
                       [pltpu.SemaphoreType.DMA] * NBUF,
                       [pltpu.SemaphoreType.DMA] * NBUF],
        compiler_params=pltpu.CompilerParams(use_tc_tiling_on_sc=False),
    )
    def k(m3_hbm, rec_hbm, out_hbm, agg_sh, zrows_v, idx2d, m3v, lsem, ssem):
        cid = lax.axis_index("c")
        sid = lax.axis_index("s")
        wid = sid * NC + cid
        base = wid * EPW

        def zero_body(r, carry):
            zrows_v[r, :] = jnp.zeros((DH,), jnp.float32)
            return carry

        lax.fori_loop(0, RPT, zero_body, 0)
        pltpu.sync_copy(zrows_v, agg_sh.at[pl.ds(sid * RPT, RPT)])
        for j in range(NCH):
            pltpu.sync_copy(rec_hbm.at[pl.ds(base + j * CB, CB)], idx2d.at[j])
        plsc.subcore_barrier()

        ld = [None] * NBUF
        sd = [None] * NBUF
        for j in range(min(NBUF, NCH)):
            ld[j] = pltpu.async_copy(
                m3_hbm.at[pl.ds(base + j * CB, CB)], m3v[j], lsem[j])
        for j in range(NCH):
            b = j % NBUF
            ld[b].wait()
            sd[b] = pltpu.async_copy(m3v[b], agg_sh.at[idx2d.at[j]],
                                     ssem[b], add=True)
            nj = j + NBUF
            if nj < NCH:
                sd[b].wait()
                ld[b] = pltpu.async_copy(
                    m3_hbm.at[pl.ds(base + nj * CB, CB)], m3v[b], lsem[b])
        for j in range(max(0, NCH - NBUF), NCH):
            sd[j % NBUF].wait()
        plsc.subcore_barrier()
        pltpu.sync_copy(agg_sh.at[pl.ds(sid * RPT, RPT)],
                        out_hbm.at[pl.ds(cid * N + sid * RPT, RPT)])

    return k(m3, rec)


def kernel(x, edge_index, edge_attr, u, batch, mw1, mb1, mw2, mb2, mw3, mb3,
           nw1, nb1, nw2, nb2, nw3, nb3):
    send = edge_index[0]
    rec = edge_index[1]
    mw1a = mw1[:, :DF]
    mw1b = mw1[:, DF:]
    nw1a = nw1[:, :DF]
    nw1b = nw1[:, DF:]
    b1 = mb1.reshape(1, DH)
    b2 = mb2.reshape(1, DH)
    b3 = mb3.reshape(1, DH)
    c1 = nb1.reshape(1, DH)
    c2 = nb2.reshape(1, DH)
    c3 = nb3.reshape(1, DH)

    xs = _node_pre(x, mw1a)
    xg = _gather_sc(xs, send)
    m3 = _edge_mlp(xg, edge_attr, mw1b, b1, mw2, b2, mw3, b3)
    parts = _scatter_sc(m3, rec)
    h = _node_mlp(x, parts, nw1a, nw1b, c1, nw2, c2, nw3, c3)
    return h
